# baseline (speedup 1.0000x reference)
import functools

import jax
import jax.numpy as jnp
from jax import lax
from jax.experimental import pallas as pl
from jax.experimental.pallas import tpu as pltpu
from jax.experimental.pallas import tpu_sc as plsc

HIDDEN = 64
K = 20
NC, NS, L = 2, 16, 16
NW = NC * NS
TOTAL_BASKETS = 1024 * 50
B_PER_W = TOTAL_BASKETS // NW
CHUNK_B = 32
N_CHUNKS = B_PER_W // CHUNK_B
IDX_PER_CHUNK = CHUNK_B * K
N_GATHERS = IDX_PER_CHUNK // 128
NVREG = HIDDEN // L


def _body(idx_hbm, table_hbm, out_hbm, idx_v, rows_v, out_v,
          gsem0, gsem1, osem0, osem1):
    wid = lax.axis_index("s") * NC + lax.axis_index("c")

    def fire_gathers(g, slot, sem):
        for j in range(N_GATHERS):
            pltpu.async_copy(
                table_hbm.at[idx_v.at[g * N_GATHERS + j]],
                rows_v.at[slot, pl.ds(j * 128, 128)],
                sem,
            )

    def wait_gathers(slot, sem):
        for j in range(N_GATHERS):
            pltpu.make_async_copy(
                table_hbm.at[idx_v.at[j]],
                rows_v.at[slot, pl.ds(j * 128, 128)],
                sem,
            ).wait()

    def compute_chunk(g, slot):
        @pl.loop(0, CHUNK_B, unroll=2)
        def basket(c):
            base = c * K
            for h in range(2):
                va, vb = [], []
                for k in range(0, K, 2):
                    a0, b0 = plsc.unpack(
                        rows_v[slot, base + k, pl.ds(h * 32, 32)],
                        format=plsc.PackFormat.INTERLEAVED)
                    a1, b1 = plsc.unpack(
                        rows_v[slot, base + k + 1, pl.ds(h * 32, 32)],
                        format=plsc.PackFormat.INTERLEAVED)
                    va.append(a0 + a1)
                    vb.append(b0 + b1)
                while len(va) > 1:
                    va = [va[i] + va[i + 1] for i in range(0, len(va) - 1, 2)] \
                        + ([va[-1]] if len(va) % 2 else [])
                    vb = [vb[i] + vb[i + 1] for i in range(0, len(vb) - 1, 2)] \
                        + ([vb[-1]] if len(vb) % 2 else [])
                scale = jnp.float32(1.0 / K)
                out_v[slot, c, pl.ds(h * 32, 32)] = plsc.pack(
                    va[0] * scale, vb[0] * scale,
                    format=plsc.PackFormat.INTERLEAVED)
        pltpu.async_copy(
            out_v.at[slot],
            out_hbm.at[pl.ds(wid * B_PER_W + g * CHUNK_B, CHUNK_B)],
            osems[slot],
        )

    def wait_out(slot):
        pltpu.make_async_copy(
            out_v.at[slot],
            out_hbm.at[pl.ds(wid * B_PER_W, CHUNK_B)],
            osems[slot],
        ).wait()

    gsems = (gsem0, gsem1)
    osems = (osem0, osem1)

    pltpu.sync_copy(idx_hbm.at[wid], idx_v)
    fire_gathers(0, 0, gsem0)

    @pl.loop(0, N_CHUNKS, step=2)
    def _chunks(g0):
        for b in range(2):
            g = g0 + b
            nxt = 1 - b
            if b == 0:
                fire_gathers(g + 1, nxt, gsems[nxt])
            else:
                @pl.when(g0 < N_CHUNKS - 2)
                def _():
                    fire_gathers(g + 1, nxt, gsems[nxt])
            wait_gathers(b, gsems[b])
            @pl.when(g >= 2)
            def _():
                wait_out(b)
            compute_chunk(g, b)

    wait_out(0)
    wait_out(1)


@jax.jit
def _pooled(idx, table):
    mesh = plsc.VectorSubcoreMesh(
        core_axis_name="c", subcore_axis_name="s",
        num_cores=NC, num_subcores=NS,
    )
    run = functools.partial(
        pl.kernel,
        out_type=jax.ShapeDtypeStruct((TOTAL_BASKETS, HIDDEN), jnp.bfloat16),
        mesh=mesh,
        compiler_params=pltpu.CompilerParams(
            use_tc_tiling_on_sc=False, needs_layout_passes=False),
        scratch_types=[
            pltpu.VMEM((N_CHUNKS * N_GATHERS, 128), jnp.int32),
            pltpu.VMEM((2, IDX_PER_CHUNK, HIDDEN), jnp.bfloat16),
            pltpu.VMEM((2, CHUNK_B, HIDDEN), jnp.bfloat16),
            pltpu.SemaphoreType.DMA,
            pltpu.SemaphoreType.DMA,
            pltpu.SemaphoreType.DMA,
            pltpu.SemaphoreType.DMA,
        ],
    )(_body)
    return run(idx, table)


def kernel(batch_basket, table):
    idx = batch_basket.reshape(NW, N_CHUNKS * N_GATHERS, 128)
    tbl = jnp.pad(table, ((0, 111), (0, 0))).astype(jnp.bfloat16)
    out = _pooled(idx, tbl)
    return out.reshape(1024, 50, HIDDEN).astype(jnp.float32)


# speedup vs baseline: 1.2055x; 1.2055x over previous
import functools

import jax
import jax.numpy as jnp
from jax import lax
from jax.experimental import pallas as pl
from jax.experimental.pallas import tpu as pltpu
from jax.experimental.pallas import tpu_sc as plsc

HIDDEN = 64
K = 20
NC, NS, L = 2, 16, 16
NW = NC * NS
TOTAL_BASKETS = 1024 * 50
B_PER_W = TOTAL_BASKETS // NW
IDX_PER_W = B_PER_W * K
CHUNK_B = 16
N_CHUNKS = B_PER_W // CHUNK_B
IDX_PER_CHUNK = CHUNK_B * K
GATHER_SPLITS = ((0, 128), (128, 128), (256, 64))
NBUF = 4
NVREG = HIDDEN // L


def _body(idx_hbm, table_hbm, out_hbm, idx_v, rows_v, out_v,
          gsem0, gsem1, gsem2, gsem3, osem0, osem1, osem2, osem3):
    wid = lax.axis_index("s") * NC + lax.axis_index("c")

    def fire_gathers(g, slot, sem):
        for off, n in GATHER_SPLITS:
            pltpu.async_copy(
                table_hbm.at[idx_v.at[pl.ds(g * IDX_PER_CHUNK + off, n)]],
                rows_v.at[slot, pl.ds(off, n)],
                sem,
            )

    def wait_gathers(slot, sem):
        for off, n in GATHER_SPLITS:
            pltpu.make_async_copy(
                table_hbm.at[idx_v.at[pl.ds(off, n)]],
                rows_v.at[slot, pl.ds(off, n)],
                sem,
            ).wait()

    def compute_chunk(g, slot):
        @pl.loop(0, CHUNK_B, unroll=2)
        def basket(c):
            base = c * K
            for j in range(NVREG):
                vs = [rows_v[slot, base + k, pl.ds(j * L, L)] +
                      rows_v[slot, base + k + 1, pl.ds(j * L, L)]
                      for k in range(0, K, 2)]
                while len(vs) > 1:
                    nxt_vs = [vs[i] + vs[i + 1] for i in range(0, len(vs) - 1, 2)]
                    if len(vs) % 2:
                        nxt_vs.append(vs[-1])
                    vs = nxt_vs
                out_v[slot, c, pl.ds(j * L, L)] = vs[0] * jnp.float32(1.0 / K)
        pltpu.async_copy(
            out_v.at[slot],
            out_hbm.at[pl.ds(wid * B_PER_W + g * CHUNK_B, CHUNK_B)],
            osems[slot],
        )

    def wait_out(slot):
        pltpu.make_async_copy(
            out_v.at[slot],
            out_hbm.at[pl.ds(wid * B_PER_W, CHUNK_B)],
            osems[slot],
        ).wait()

    gsems = (gsem0, gsem1, gsem2, gsem3)
    osems = (osem0, osem1, osem2, osem3)

    pltpu.sync_copy(idx_hbm.at[wid], idx_v)
    for g in range(NBUF - 1):
        fire_gathers(g, g, gsems[g])

    @pl.loop(0, N_CHUNKS, step=NBUF)
    def _chunks(g0):
        for b in range(NBUF):
            g = g0 + b
            pre = (b + NBUF - 1) % NBUF
            if b == 0:
                fire_gathers(g + NBUF - 1, pre, gsems[pre])
            else:
                @pl.when(g0 < N_CHUNKS - NBUF)
                def _():
                    fire_gathers(g + NBUF - 1, pre, gsems[pre])
            wait_gathers(b, gsems[b])
            @pl.when(g >= NBUF)
            def _():
                wait_out(b)
            compute_chunk(g, b)

    for b in range(NBUF):
        wait_out(b)


@jax.jit
def _pooled(idx, table):
    mesh = plsc.VectorSubcoreMesh(
        core_axis_name="c", subcore_axis_name="s",
        num_cores=NC, num_subcores=NS,
    )
    run = functools.partial(
        pl.kernel,
        out_type=jax.ShapeDtypeStruct((TOTAL_BASKETS, HIDDEN), jnp.float32),
        mesh=mesh,
        compiler_params=pltpu.CompilerParams(use_tc_tiling_on_sc=False),
        scratch_types=[
            pltpu.VMEM((IDX_PER_W,), jnp.int32),
            pltpu.VMEM((NBUF, IDX_PER_CHUNK, HIDDEN), jnp.float32),
            pltpu.VMEM((NBUF, CHUNK_B, HIDDEN), jnp.float32),
            pltpu.SemaphoreType.DMA,
            pltpu.SemaphoreType.DMA,
            pltpu.SemaphoreType.DMA,
            pltpu.SemaphoreType.DMA,
            pltpu.SemaphoreType.DMA,
            pltpu.SemaphoreType.DMA,
            pltpu.SemaphoreType.DMA,
            pltpu.SemaphoreType.DMA,
        ],
    )(_body)
    return run(idx, table)


def kernel(batch_basket, table):
    idx = batch_basket.reshape(NW, IDX_PER_W)
    out = _pooled(idx, table)
    return out.reshape(1024, 50, HIDDEN)
